# baseline (speedup 1.0000x reference)
import functools

import jax
import jax.numpy as jnp
from jax import lax
from jax.experimental import pallas as pl
from jax.experimental.pallas import tpu as pltpu
from jax.experimental.pallas import tpu_sc as plsc

N_NODES = 10000
N_EDGES = 160000
D = 128
EPS = 1e-5
DT = 0.01

BN = 2000
BE = 8000
CHUNK = 128
NCH = N_EDGES // CHUNK
NC = 2
NS = 16
NW = NC * NS
ROWS_PER_TILE = N_NODES // NS


def _dot(a, b):
    return jnp.dot(a, b, preferred_element_type=jnp.float32)


def _ln_fwd(u):
    mu = jnp.mean(u, axis=-1, keepdims=True)
    var = jnp.mean((u - mu) * (u - mu), axis=-1, keepdims=True)
    inv = lax.rsqrt(var + EPS)
    return (u - mu) * inv, inv


def _ln_bwd(g, y, inv):
    return inv * (g - jnp.mean(g, axis=-1, keepdims=True)
                  - y * jnp.mean(g * y, axis=-1, keepdims=True))


def _node_fwd_body(x_r, wn0_r, bn0_r, wn1_r, bn1_r, ws_r, wd_r,
                   a_r, b_r, u_r, z_r):
    z = _dot(x_r[...], wn0_r[...]) + bn0_r[...]
    r = jnp.maximum(z, 0.0)
    u = _dot(r, wn1_r[...]) + bn1_r[...]
    y, _ = _ln_fwd(u)
    a_r[...] = _dot(y, ws_r[...])
    b_r[...] = _dot(y, wd_r[...])
    u_r[...] = u
    z_r[...] = z


def _node_fwd(x, wn0, bn0, wn1, bn1, wsrc, wdst):
    grid = (N_NODES // BN,)
    row = pl.BlockSpec((BN, D), lambda i: (i, 0))
    w = pl.BlockSpec((D, D), lambda i: (0, 0))
    bsp = pl.BlockSpec((1, D), lambda i: (0, 0))
    out = jax.ShapeDtypeStruct((N_NODES, D), jnp.float32)
    return pl.pallas_call(
        _node_fwd_body,
        grid=grid,
        in_specs=[row, w, bsp, w, bsp, w, w],
        out_specs=[row, row, row, row],
        out_shape=[out, out, out, out],
    )(x, wn0, bn0, wn1, bn1, wsrc, wdst)


def _edge_body(ea_r, ga_r, gb_r, we0_r, be0_r, we1_r, be1_r, wme_r, bm0_r,
               wm1_r, bm1_r, wm1t_r, wd0_r, bd0_r, wd0t_r, wd1_r, gz_r):
    he0p = _dot(jnp.maximum(_dot(ea_r[...], we0_r[...]) + be0_r[...], 0.0),
                we1_r[...]) + be1_r[...]
    he0, _ = _ln_fwd(he0p)
    z = _dot(he0, wme_r[...]) + ga_r[...] + gb_r[...] + bm0_r[...]
    r = jnp.maximum(z, 0.0)
    u = _dot(r, wm1_r[...]) + bm1_r[...]
    m, inv2 = _ln_fwd(u)
    he = he0 + m
    dpre = _dot(he, wd0_r[...]) + bd0_r[...]
    gdpre = jnp.where(dpre > 0.0, wd1_r[...], 0.0)
    ghe = _dot(gdpre, wd0t_r[...])
    gu = _ln_bwd(ghe, m, inv2)
    gr = _dot(gu, wm1t_r[...])
    gz_r[...] = jnp.where(z > 0.0, gr, 0.0)


def _edge_pipe(ea, ga, gb, we0, be0, we1, be1, wme, bm0, wm1, bm1, wm1t,
               wd0, bd0, wd0t, wd1row):
    ne = ea.shape[0]
    grid = (ne // BE,)
    row = pl.BlockSpec((BE, D), lambda i: (i, 0))
    ea_sp = pl.BlockSpec((BE, 16), lambda i: (i, 0))
    w = pl.BlockSpec((D, D), lambda i: (0, 0))
    w16 = pl.BlockSpec((16, D), lambda i: (0, 0))
    bsp = pl.BlockSpec((1, D), lambda i: (0, 0))
    return pl.pallas_call(
        _edge_body,
        grid=grid,
        in_specs=[ea_sp, row, row, w16, bsp, w, bsp, w, bsp, w, bsp, w, w,
                  bsp, w, bsp],
        out_specs=row,
        out_shape=jax.ShapeDtypeStruct((ne, D), jnp.float32),
    )(ea, ga, gb, we0, be0, we1, be1, wme, bm0, wm1, bm1, wm1t, wd0, bd0,
      wd0t, wd1row)


def _node_bwd_body(x_r, z_r, u_r, gs_r, gd_r, wst_r, wdt_r,
                   wn1t_r, wn0t_r, o_r):
    ghv = _dot(gs_r[...], wst_r[...]) + _dot(gd_r[...], wdt_r[...])
    y, inv = _ln_fwd(u_r[...])
    guv = _ln_bwd(ghv, y, inv)
    grv = _dot(guv, wn1t_r[...])
    gzv = jnp.where(z_r[...] > 0.0, grv, 0.0)
    o_r[...] = x_r[...] - DT * _dot(gzv, wn0t_r[...])


def _node_bwd(x, z, u, gs, gd, wst, wdt, wn1t, wn0t):
    grid = (N_NODES // BN,)
    row = pl.BlockSpec((BN, D), lambda i: (i, 0))
    w = pl.BlockSpec((D, D), lambda i: (0, 0))
    return pl.pallas_call(
        _node_bwd_body,
        grid=grid,
        in_specs=[row, row, row, row, row, w, w, w, w],
        out_specs=row,
        out_shape=jax.ShapeDtypeStruct((N_NODES, D), jnp.float32),
    )(x, z, u, gs, gd, wst, wdt, wn1t, wn0t)


MAXCH = 40
IDXBUF = MAXCH * CHUNK
GBUF = 3
SBUF = 3


def _sc_gather(a, b, src, dst):
    ne = src.shape[0]
    nch = ne // CHUNK
    mesh = plsc.VectorSubcoreMesh(core_axis_name="c", subcore_axis_name="s")
    out = jax.ShapeDtypeStruct((ne, D), jnp.float32)

    @functools.partial(
        pl.kernel,
        out_type=[out, out],
        mesh=mesh,
        scratch_types=[
            pltpu.VMEM((GBUF, CHUNK), jnp.int32),
            pltpu.VMEM((GBUF, CHUNK, D), jnp.float32),
            pltpu.VMEM_SHARED((N_NODES, D), jnp.float32),
        ] + [pltpu.SemaphoreType.DMA] * (3 * GBUF),
    )
    def k(a_h, b_h, src_h, dst_h, oa_h, ob_h, idx, rows, tbl, *sems):
        si = sems[:GBUF]
        sg = sems[GBUF:2 * GBUF]
        sw = sems[2 * GBUF:]
        c = lax.axis_index("c")
        s = lax.axis_index("s")

        rbase = s * 1000

        @pl.when(jnp.logical_and(c == 0, s < 10))
        def _():
            pltpu.sync_copy(a_h.at[pl.ds(rbase, 1000)],
                            tbl.at[pl.ds(rbase, 1000)])

        @pl.when(jnp.logical_and(c == 1, s < 10))
        def _():
            pltpu.sync_copy(b_h.at[pl.ds(rbase, 1000)],
                            tbl.at[pl.ds(rbase, 1000)])

        plsc.subcore_barrier()

        lo = (s * nch) // NS
        hi = ((s + 1) * nch) // NS

        def i_issue(cn, p):
            base = cn * CHUNK

            @pl.when(c == 0)
            def _():
                pltpu.async_copy(src_h.at[pl.ds(base, CHUNK)], idx.at[p],
                                 si[p])

            @pl.when(c == 1)
            def _():
                pltpu.async_copy(dst_h.at[pl.ds(base, CHUNK)], idx.at[p],
                                 si[p])

        def i_wait(p):
            pltpu.make_async_copy(src_h.at[pl.ds(0, CHUNK)], idx.at[p],
                                  si[p]).wait()

        def g_issue(p):
            pltpu.async_copy(tbl.at[idx.at[p]], rows.at[p], sg[p])

        def g_wait(p):
            pltpu.make_async_copy(tbl.at[idx.at[p]], rows.at[p],
                                  sg[p]).wait()

        def w_issue(cn, p):
            base = cn * CHUNK

            @pl.when(c == 0)
            def _():
                pltpu.async_copy(rows.at[p], oa_h.at[pl.ds(base, CHUNK)],
                                 sw[p])

            @pl.when(c == 1)
            def _():
                pltpu.async_copy(rows.at[p], ob_h.at[pl.ds(base, CHUNK)],
                                 sw[p])

        def w_wait(p):
            pltpu.make_async_copy(rows.at[p], oa_h.at[pl.ds(0, CHUNK)],
                                  sw[p]).wait()

        for p in range(GBUF):
            @pl.when(lo + p < hi)
            def _(p=p):
                i_issue(lo + p, p)

        def body(t, carry):
            cbase = lo + GBUF * t
            for p in range(GBUF):
                @pl.when(cbase + p < hi)
                def _(p=p):
                    i_wait(p)
                    g_issue(p)
            for p in range(GBUF):
                @pl.when(cbase + p < hi)
                def _(p=p, cc=cbase + p):
                    g_wait(p)
                    w_issue(cc, p)
            for p in range(GBUF):
                @pl.when(cbase + p < hi)
                def _(p=p):
                    w_wait(p)

                @pl.when(cbase + p + GBUF < hi)
                def _(p=p, cn=cbase + p + GBUF):
                    i_issue(cn, p)
            return carry

        nloc = (nch + NS - 1) // NS
        lax.fori_loop(0, (nloc + GBUF - 1) // GBUF, body, 0)

    return k(a, b, src, dst)


def _sc_scatter(gz, src, dst, init_s, init_d):
    ne = src.shape[0]
    nch = ne // CHUNK
    mesh = plsc.VectorSubcoreMesh(core_axis_name="c", subcore_axis_name="s")
    out = jax.ShapeDtypeStruct((N_NODES, D), jnp.float32)

    @functools.partial(
        pl.kernel,
        out_type=[out, out],
        mesh=mesh,
        scratch_types=[
            pltpu.VMEM((SBUF, CHUNK), jnp.int32),
            pltpu.VMEM((SBUF, CHUNK, D), jnp.float32),
            pltpu.VMEM_SHARED((N_NODES, D), jnp.float32),
        ] + [pltpu.SemaphoreType.DMA] * (2 * SBUF),
    )
    def k(gz_h, src_h, dst_h, is_h, id_h, os_h, od_h, idx, rows, table,
          *sems):
        sl = sems[:SBUF]
        ss = sems[SBUF:]
        c = lax.axis_index("c")
        s = lax.axis_index("s")
        rbase = s * 1000

        @pl.when(jnp.logical_and(c == 0, s < 10))
        def _():
            pltpu.sync_copy(is_h.at[pl.ds(rbase, 1000)],
                            table.at[pl.ds(rbase, 1000)])

        @pl.when(jnp.logical_and(c == 1, s < 10))
        def _():
            pltpu.sync_copy(id_h.at[pl.ds(rbase, 1000)],
                            table.at[pl.ds(rbase, 1000)])

        plsc.subcore_barrier()

        lo = (s * nch) // NS
        hi = ((s + 1) * nch) // NS

        def l_issue(cn, p):
            base = cn * CHUNK

            @pl.when(c == 0)
            def _():
                pltpu.async_copy(src_h.at[pl.ds(base, CHUNK)], idx.at[p],
                                 sl[p])

            @pl.when(c == 1)
            def _():
                pltpu.async_copy(dst_h.at[pl.ds(base, CHUNK)], idx.at[p],
                                 sl[p])

            pltpu.async_copy(gz_h.at[pl.ds(base, CHUNK)], rows.at[p], sl[p])

        def l_wait(p):
            pltpu.make_async_copy(src_h.at[pl.ds(0, CHUNK)], idx.at[p],
                                  sl[p]).wait()
            pltpu.make_async_copy(gz_h.at[pl.ds(0, CHUNK)], rows.at[p],
                                  sl[p]).wait()

        def s_issue(p):
            pltpu.async_copy(rows.at[p], table.at[idx.at[p]], ss[p], add=True)

        def s_wait(p):
            pltpu.make_async_copy(rows.at[p], table.at[idx.at[p]],
                                  ss[p]).wait()

        for p in range(SBUF):
            @pl.when(lo + p < hi)
            def _(p=p):
                l_issue(lo + p, p)

        def body(t, carry):
            cbase = lo + SBUF * t
            for p in range(SBUF):
                @pl.when(cbase + p < hi)
                def _(p=p):
                    l_wait(p)
                    s_issue(p)
            for p in range(SBUF):
                @pl.when(cbase + p < hi)
                def _(p=p):
                    s_wait(p)

                @pl.when(cbase + p + SBUF < hi)
                def _(p=p, cn=cbase + p + SBUF):
                    l_issue(cn, p)
            return carry

        lax.fori_loop(0, (nch // NS + 1 + SBUF) // SBUF, body, 0)
        plsc.subcore_barrier()

        @pl.when(jnp.logical_and(c == 0, s < 10))
        def _():
            pltpu.sync_copy(table.at[pl.ds(rbase, 1000)],
                            os_h.at[pl.ds(rbase, 1000)])

        @pl.when(jnp.logical_and(c == 1, s < 10))
        def _():
            pltpu.sync_copy(table.at[pl.ds(rbase, 1000)],
                            od_h.at[pl.ds(rbase, 1000)])

    return k(gz, src, dst, init_s, init_d)


def kernel(x, edge_attr, edge_index, enc_node_w0, enc_node_b0, enc_node_w1,
           enc_node_b1, enc_edge_w0, enc_edge_b0, enc_edge_w1, enc_edge_b1,
           mp_edge_w0, mp_edge_b0, mp_edge_w1, mp_edge_b1, mp_node_w0,
           mp_node_b0, mp_node_w1, mp_node_b1, dec_edge_w0, dec_edge_b0,
           dec_edge_w1, dec_edge_b1):
    src = edge_index[0].astype(jnp.int32)
    dst = edge_index[1].astype(jnp.int32)

    wsrc = mp_edge_w0[D:2 * D]
    wdst = mp_edge_w0[2 * D:3 * D]
    wme = mp_edge_w0[0:D]

    r1 = lambda v: v.reshape(1, D)
    a, b, u_v, z_v = _node_fwd(x, enc_node_w0, r1(enc_node_b0), enc_node_w1,
                               r1(enc_node_b1), wsrc, wdst)

    edge_w = (enc_edge_w0, r1(enc_edge_b0), enc_edge_w1, r1(enc_edge_b1),
              wme, r1(mp_edge_b0), mp_edge_w1, r1(mp_edge_b1), mp_edge_w1.T,
              dec_edge_w0, r1(dec_edge_b0), dec_edge_w0.T,
              dec_edge_w1[:, 0].reshape(1, D))

    gs = jnp.zeros((N_NODES, D), jnp.float32)
    gd = gs
    bounds = [0, 48000, 96000, 128000, 160000]
    for i in range(len(bounds) - 1):
        sl = slice(bounds[i], bounds[i + 1])
        ga_i, gb_i = _sc_gather(a, b, src[sl], dst[sl])
        gz_i = _edge_pipe(edge_attr[sl], ga_i, gb_i, *edge_w)
        gs, gd = _sc_scatter(gz_i, src[sl], dst[sl], gs, gd)

    return _node_bwd(x, z_v, u_v, gs, gd, wsrc.T, wdst.T,
                     enc_node_w1.T, enc_node_w0.T)


# speedup vs baseline: 1.0310x; 1.0310x over previous
import functools

import jax
import jax.numpy as jnp
from jax import lax
from jax.experimental import pallas as pl
from jax.experimental.pallas import tpu as pltpu
from jax.experimental.pallas import tpu_sc as plsc

N_NODES = 10000
N_EDGES = 160000
D = 128
EPS = 1e-5
DT = 0.01

BN = 2000
BE = 8000
CHUNK = 128
NCH = N_EDGES // CHUNK
NC = 2
NS = 16
NW = NC * NS
ROWS_PER_TILE = N_NODES // NS


def _dot(a, b):
    return jnp.dot(a, b, preferred_element_type=jnp.float32)


def _ln_fwd(u):
    mu = jnp.mean(u, axis=-1, keepdims=True)
    var = jnp.mean((u - mu) * (u - mu), axis=-1, keepdims=True)
    inv = lax.rsqrt(var + EPS)
    return (u - mu) * inv, inv


def _ln_bwd(g, y, inv):
    return inv * (g - jnp.mean(g, axis=-1, keepdims=True)
                  - y * jnp.mean(g * y, axis=-1, keepdims=True))


def _node_fwd_body(x_r, wn0_r, bn0_r, wn1_r, bn1_r, ws_r, wd_r,
                   a_r, b_r, u_r, z_r):
    z = _dot(x_r[...], wn0_r[...]) + bn0_r[...]
    r = jnp.maximum(z, 0.0)
    u = _dot(r, wn1_r[...]) + bn1_r[...]
    y, _ = _ln_fwd(u)
    a_r[...] = _dot(y, ws_r[...])
    b_r[...] = _dot(y, wd_r[...])
    u_r[...] = u
    z_r[...] = z


def _node_fwd(x, wn0, bn0, wn1, bn1, wsrc, wdst):
    grid = (N_NODES // BN,)
    row = pl.BlockSpec((BN, D), lambda i: (i, 0))
    w = pl.BlockSpec((D, D), lambda i: (0, 0))
    bsp = pl.BlockSpec((1, D), lambda i: (0, 0))
    out = jax.ShapeDtypeStruct((N_NODES, D), jnp.float32)
    return pl.pallas_call(
        _node_fwd_body,
        grid=grid,
        in_specs=[row, w, bsp, w, bsp, w, w],
        out_specs=[row, row, row, row],
        out_shape=[out, out, out, out],
    )(x, wn0, bn0, wn1, bn1, wsrc, wdst)


def _edge_body(ea_r, ga_r, gb_r, we0_r, be0_r, we1_r, be1_r, wme_r, bm0_r,
               wm1_r, bm1_r, wm1t_r, wd0_r, bd0_r, wd0t_r, wd1_r, gz_r):
    he0p = _dot(jnp.maximum(_dot(ea_r[...], we0_r[...]) + be0_r[...], 0.0),
                we1_r[...]) + be1_r[...]
    he0, _ = _ln_fwd(he0p)
    z = _dot(he0, wme_r[...]) + ga_r[...] + gb_r[...] + bm0_r[...]
    r = jnp.maximum(z, 0.0)
    u = _dot(r, wm1_r[...]) + bm1_r[...]
    m, inv2 = _ln_fwd(u)
    he = he0 + m
    dpre = _dot(he, wd0_r[...]) + bd0_r[...]
    gdpre = jnp.where(dpre > 0.0, wd1_r[...], 0.0)
    ghe = _dot(gdpre, wd0t_r[...])
    gu = _ln_bwd(ghe, m, inv2)
    gr = _dot(gu, wm1t_r[...])
    gz_r[...] = jnp.where(z > 0.0, gr, 0.0)


def _edge_pipe(ea, ga, gb, we0, be0, we1, be1, wme, bm0, wm1, bm1, wm1t,
               wd0, bd0, wd0t, wd1row):
    ne = ea.shape[0]
    grid = (ne // BE,)
    row = pl.BlockSpec((BE, D), lambda i: (i, 0))
    ea_sp = pl.BlockSpec((BE, 16), lambda i: (i, 0))
    w = pl.BlockSpec((D, D), lambda i: (0, 0))
    w16 = pl.BlockSpec((16, D), lambda i: (0, 0))
    bsp = pl.BlockSpec((1, D), lambda i: (0, 0))
    return pl.pallas_call(
        _edge_body,
        grid=grid,
        in_specs=[ea_sp, row, row, w16, bsp, w, bsp, w, bsp, w, bsp, w, w,
                  bsp, w, bsp],
        out_specs=row,
        out_shape=jax.ShapeDtypeStruct((ne, D), jnp.float32),
    )(ea, ga, gb, we0, be0, we1, be1, wme, bm0, wm1, bm1, wm1t, wd0, bd0,
      wd0t, wd1row)


def _node_bwd_body(x_r, z_r, u_r, gs_r, gd_r, wst_r, wdt_r,
                   wn1t_r, wn0t_r, o_r):
    ghv = _dot(gs_r[...], wst_r[...]) + _dot(gd_r[...], wdt_r[...])
    y, inv = _ln_fwd(u_r[...])
    guv = _ln_bwd(ghv, y, inv)
    grv = _dot(guv, wn1t_r[...])
    gzv = jnp.where(z_r[...] > 0.0, grv, 0.0)
    o_r[...] = x_r[...] - DT * _dot(gzv, wn0t_r[...])


def _node_bwd(x, z, u, gs, gd, wst, wdt, wn1t, wn0t):
    grid = (N_NODES // BN,)
    row = pl.BlockSpec((BN, D), lambda i: (i, 0))
    w = pl.BlockSpec((D, D), lambda i: (0, 0))
    return pl.pallas_call(
        _node_bwd_body,
        grid=grid,
        in_specs=[row, row, row, row, row, w, w, w, w],
        out_specs=row,
        out_shape=jax.ShapeDtypeStruct((N_NODES, D), jnp.float32),
    )(x, z, u, gs, gd, wst, wdt, wn1t, wn0t)


MAXCH = 40
IDXBUF = MAXCH * CHUNK
GBUF = 3
SBUF = 3


def _sc_gather(a, b, src, dst):
    ne = src.shape[0]
    nch = ne // CHUNK
    mesh = plsc.VectorSubcoreMesh(core_axis_name="c", subcore_axis_name="s")
    out = jax.ShapeDtypeStruct((ne, D), jnp.float32)

    @functools.partial(
        pl.kernel,
        out_type=[out, out],
        mesh=mesh,
        scratch_types=[
            pltpu.VMEM((GBUF, CHUNK), jnp.int32),
            pltpu.VMEM((GBUF, CHUNK, D), jnp.float32),
            pltpu.VMEM_SHARED((N_NODES, D), jnp.float32),
        ] + [pltpu.SemaphoreType.DMA] * (3 * GBUF),
    )
    def k(a_h, b_h, src_h, dst_h, oa_h, ob_h, idx, rows, tbl, *sems):
        si = sems[:GBUF]
        sg = sems[GBUF:2 * GBUF]
        sw = sems[2 * GBUF:]
        c = lax.axis_index("c")
        s = lax.axis_index("s")

        rbase = s * 1000

        @pl.when(jnp.logical_and(c == 0, s < 10))
        def _():
            pltpu.sync_copy(a_h.at[pl.ds(rbase, 1000)],
                            tbl.at[pl.ds(rbase, 1000)])

        @pl.when(jnp.logical_and(c == 1, s < 10))
        def _():
            pltpu.sync_copy(b_h.at[pl.ds(rbase, 1000)],
                            tbl.at[pl.ds(rbase, 1000)])

        plsc.subcore_barrier()

        lo = (s * nch) // NS
        hi = ((s + 1) * nch) // NS

        def i_issue(cn, p):
            base = cn * CHUNK

            @pl.when(c == 0)
            def _():
                pltpu.async_copy(src_h.at[pl.ds(base, CHUNK)], idx.at[p],
                                 si[p])

            @pl.when(c == 1)
            def _():
                pltpu.async_copy(dst_h.at[pl.ds(base, CHUNK)], idx.at[p],
                                 si[p])

        def i_wait(p):
            pltpu.make_async_copy(src_h.at[pl.ds(0, CHUNK)], idx.at[p],
                                  si[p]).wait()

        def g_issue(p):
            pltpu.async_copy(tbl.at[idx.at[p]], rows.at[p], sg[p])

        def g_wait(p):
            pltpu.make_async_copy(tbl.at[idx.at[p]], rows.at[p],
                                  sg[p]).wait()

        def w_issue(cn, p):
            base = cn * CHUNK

            @pl.when(c == 0)
            def _():
                pltpu.async_copy(rows.at[p], oa_h.at[pl.ds(base, CHUNK)],
                                 sw[p])

            @pl.when(c == 1)
            def _():
                pltpu.async_copy(rows.at[p], ob_h.at[pl.ds(base, CHUNK)],
                                 sw[p])

        def w_wait(p):
            pltpu.make_async_copy(rows.at[p], oa_h.at[pl.ds(0, CHUNK)],
                                  sw[p]).wait()

        for p in range(GBUF):
            @pl.when(lo + p < hi)
            def _(p=p):
                i_issue(lo + p, p)

        def body(t, carry):
            cbase = lo + GBUF * t
            for p in range(GBUF):
                @pl.when(cbase + p < hi)
                def _(p=p):
                    i_wait(p)
                    g_issue(p)
            for p in range(GBUF):
                @pl.when(cbase + p < hi)
                def _(p=p, cc=cbase + p):
                    g_wait(p)
                    w_issue(cc, p)
            for p in range(GBUF):
                @pl.when(cbase + p < hi)
                def _(p=p):
                    w_wait(p)

                @pl.when(cbase + p + GBUF < hi)
                def _(p=p, cn=cbase + p + GBUF):
                    i_issue(cn, p)
            return carry

        nloc = (nch + NS - 1) // NS
        lax.fori_loop(0, (nloc + GBUF - 1) // GBUF, body, 0)

    return k(a, b, src, dst)


def _sc_scatter(gz, src, dst, init_s, init_d):
    ne = src.shape[0]
    nch = ne // CHUNK
    mesh = plsc.VectorSubcoreMesh(core_axis_name="c", subcore_axis_name="s")
    out = jax.ShapeDtypeStruct((N_NODES, D), jnp.float32)

    @functools.partial(
        pl.kernel,
        out_type=[out, out],
        mesh=mesh,
        scratch_types=[
            pltpu.VMEM((SBUF, CHUNK), jnp.int32),
            pltpu.VMEM((SBUF, CHUNK, D), jnp.float32),
            pltpu.VMEM_SHARED((N_NODES, D), jnp.float32),
        ] + [pltpu.SemaphoreType.DMA] * (2 * SBUF),
    )
    def k(gz_h, src_h, dst_h, is_h, id_h, os_h, od_h, idx, rows, table,
          *sems):
        sl = sems[:SBUF]
        ss = sems[SBUF:]
        c = lax.axis_index("c")
        s = lax.axis_index("s")
        rbase = s * 1000

        @pl.when(jnp.logical_and(c == 0, s < 10))
        def _():
            pltpu.sync_copy(is_h.at[pl.ds(rbase, 1000)],
                            table.at[pl.ds(rbase, 1000)])

        @pl.when(jnp.logical_and(c == 1, s < 10))
        def _():
            pltpu.sync_copy(id_h.at[pl.ds(rbase, 1000)],
                            table.at[pl.ds(rbase, 1000)])

        plsc.subcore_barrier()

        lo = (s * nch) // NS
        hi = ((s + 1) * nch) // NS

        def l_issue(cn, p):
            base = cn * CHUNK

            @pl.when(c == 0)
            def _():
                pltpu.async_copy(src_h.at[pl.ds(base, CHUNK)], idx.at[p],
                                 sl[p])

            @pl.when(c == 1)
            def _():
                pltpu.async_copy(dst_h.at[pl.ds(base, CHUNK)], idx.at[p],
                                 sl[p])

            pltpu.async_copy(gz_h.at[pl.ds(base, CHUNK)], rows.at[p], sl[p])

        def l_wait(p):
            pltpu.make_async_copy(src_h.at[pl.ds(0, CHUNK)], idx.at[p],
                                  sl[p]).wait()
            pltpu.make_async_copy(gz_h.at[pl.ds(0, CHUNK)], rows.at[p],
                                  sl[p]).wait()

        def s_issue(p):
            pltpu.async_copy(rows.at[p], table.at[idx.at[p]], ss[p], add=True)

        def s_wait(p):
            pltpu.make_async_copy(rows.at[p], table.at[idx.at[p]],
                                  ss[p]).wait()

        for p in range(SBUF):
            @pl.when(lo + p < hi)
            def _(p=p):
                l_issue(lo + p, p)

        def body(t, carry):
            cbase = lo + SBUF * t
            for p in range(SBUF):
                @pl.when(cbase + p < hi)
                def _(p=p):
                    l_wait(p)
                    s_issue(p)
            for p in range(SBUF):
                @pl.when(cbase + p < hi)
                def _(p=p):
                    s_wait(p)

                @pl.when(cbase + p + SBUF < hi)
                def _(p=p, cn=cbase + p + SBUF):
                    l_issue(cn, p)
            return carry

        lax.fori_loop(0, (nch // NS + 1 + SBUF) // SBUF, body, 0)
        plsc.subcore_barrier()

        @pl.when(jnp.logical_and(c == 0, s < 10))
        def _():
            pltpu.sync_copy(table.at[pl.ds(rbase, 1000)],
                            os_h.at[pl.ds(rbase, 1000)])

        @pl.when(jnp.logical_and(c == 1, s < 10))
        def _():
            pltpu.sync_copy(table.at[pl.ds(rbase, 1000)],
                            od_h.at[pl.ds(rbase, 1000)])

    return k(gz, src, dst, init_s, init_d)


def kernel(x, edge_attr, edge_index, enc_node_w0, enc_node_b0, enc_node_w1,
           enc_node_b1, enc_edge_w0, enc_edge_b0, enc_edge_w1, enc_edge_b1,
           mp_edge_w0, mp_edge_b0, mp_edge_w1, mp_edge_b1, mp_node_w0,
           mp_node_b0, mp_node_w1, mp_node_b1, dec_edge_w0, dec_edge_b0,
           dec_edge_w1, dec_edge_b1):
    src = edge_index[0].astype(jnp.int32)
    dst = edge_index[1].astype(jnp.int32)

    wsrc = mp_edge_w0[D:2 * D]
    wdst = mp_edge_w0[2 * D:3 * D]
    wme = mp_edge_w0[0:D]

    r1 = lambda v: v.reshape(1, D)
    a, b, u_v, z_v = _node_fwd(x, enc_node_w0, r1(enc_node_b0), enc_node_w1,
                               r1(enc_node_b1), wsrc, wdst)

    edge_w = (enc_edge_w0, r1(enc_edge_b0), enc_edge_w1, r1(enc_edge_b1),
              wme, r1(mp_edge_b0), mp_edge_w1, r1(mp_edge_b1), mp_edge_w1.T,
              dec_edge_w0, r1(dec_edge_b0), dec_edge_w0.T,
              dec_edge_w1[:, 0].reshape(1, D))

    gs = jnp.zeros((N_NODES, D), jnp.float32)
    gd = gs
    bounds = [0, 80000, 128000, 160000]
    for i in range(len(bounds) - 1):
        sl = slice(bounds[i], bounds[i + 1])
        ga_i, gb_i = _sc_gather(a, b, src[sl], dst[sl])
        gz_i = _edge_pipe(edge_attr[sl], ga_i, gb_i, *edge_w)
        gs, gd = _sc_scatter(gz_i, src[sl], dst[sl], gs, gd)

    return _node_bwd(x, z_v, u_v, gs, gd, wsrc.T, wdst.T,
                     enc_node_w1.T, enc_node_w0.T)


# speedup vs baseline: 1.0686x; 1.0365x over previous
import functools

import jax
import jax.numpy as jnp
from jax import lax
from jax.experimental import pallas as pl
from jax.experimental.pallas import tpu as pltpu
from jax.experimental.pallas import tpu_sc as plsc

N_NODES = 10000
N_EDGES = 160000
D = 128
EPS = 1e-5
DT = 0.01

BN = 2000
BE = 8000
CHUNK = 128
NCH = N_EDGES // CHUNK
NC = 2
NS = 16
NW = NC * NS
ROWS_PER_TILE = N_NODES // NS


def _dot(a, b):
    return jnp.dot(a, b, preferred_element_type=jnp.float32)


def _ln_fwd(u):
    mu = jnp.mean(u, axis=-1, keepdims=True)
    var = jnp.mean((u - mu) * (u - mu), axis=-1, keepdims=True)
    inv = lax.rsqrt(var + EPS)
    return (u - mu) * inv, inv


def _ln_bwd(g, y, inv):
    return inv * (g - jnp.mean(g, axis=-1, keepdims=True)
                  - y * jnp.mean(g * y, axis=-1, keepdims=True))


def _node_fwd_body(x_r, wn0_r, bn0_r, wn1_r, bn1_r, ws_r, wd_r,
                   a_r, b_r, u_r, z_r):
    z = _dot(x_r[...], wn0_r[...]) + bn0_r[...]
    r = jnp.maximum(z, 0.0)
    u = _dot(r, wn1_r[...]) + bn1_r[...]
    y, _ = _ln_fwd(u)
    a_r[...] = _dot(y, ws_r[...])
    b_r[...] = _dot(y, wd_r[...])
    u_r[...] = u
    z_r[...] = z


def _node_fwd(x, wn0, bn0, wn1, bn1, wsrc, wdst):
    grid = (N_NODES // BN,)
    row = pl.BlockSpec((BN, D), lambda i: (i, 0))
    w = pl.BlockSpec((D, D), lambda i: (0, 0))
    bsp = pl.BlockSpec((1, D), lambda i: (0, 0))
    out = jax.ShapeDtypeStruct((N_NODES, D), jnp.float32)
    return pl.pallas_call(
        _node_fwd_body,
        grid=grid,
        in_specs=[row, w, bsp, w, bsp, w, w],
        out_specs=[row, row, row, row],
        out_shape=[out, out, out, out],
    )(x, wn0, bn0, wn1, bn1, wsrc, wdst)


def _edge_body(ea_r, ga_r, gb_r, we0_r, be0_r, we1_r, be1_r, wme_r, bm0_r,
               wm1_r, bm1_r, wm1t_r, wd0_r, bd0_r, wd0t_r, wd1_r, gz_r):
    he0p = _dot(jnp.maximum(_dot(ea_r[...], we0_r[...]) + be0_r[...], 0.0),
                we1_r[...]) + be1_r[...]
    he0, _ = _ln_fwd(he0p)
    z = _dot(he0, wme_r[...]) + ga_r[...] + gb_r[...] + bm0_r[...]
    r = jnp.maximum(z, 0.0)
    u = _dot(r, wm1_r[...]) + bm1_r[...]
    m, inv2 = _ln_fwd(u)
    he = he0 + m
    dpre = _dot(he, wd0_r[...]) + bd0_r[...]
    gdpre = jnp.where(dpre > 0.0, wd1_r[...], 0.0)
    ghe = _dot(gdpre, wd0t_r[...])
    gu = _ln_bwd(ghe, m, inv2)
    gr = _dot(gu, wm1t_r[...])
    gz_r[...] = jnp.where(z > 0.0, gr, 0.0)


def _edge_pipe(ea, ga, gb, we0, be0, we1, be1, wme, bm0, wm1, bm1, wm1t,
               wd0, bd0, wd0t, wd1row):
    ne = ea.shape[0]
    grid = (ne // BE,)
    row = pl.BlockSpec((BE, D), lambda i: (i, 0))
    ea_sp = pl.BlockSpec((BE, 16), lambda i: (i, 0))
    w = pl.BlockSpec((D, D), lambda i: (0, 0))
    w16 = pl.BlockSpec((16, D), lambda i: (0, 0))
    bsp = pl.BlockSpec((1, D), lambda i: (0, 0))
    return pl.pallas_call(
        _edge_body,
        grid=grid,
        in_specs=[ea_sp, row, row, w16, bsp, w, bsp, w, bsp, w, bsp, w, w,
                  bsp, w, bsp],
        out_specs=row,
        out_shape=jax.ShapeDtypeStruct((ne, D), jnp.float32),
    )(ea, ga, gb, we0, be0, we1, be1, wme, bm0, wm1, bm1, wm1t, wd0, bd0,
      wd0t, wd1row)


def _node_bwd_body(x_r, z_r, u_r, gs_r, gd_r, wst_r, wdt_r,
                   wn1t_r, wn0t_r, o_r):
    ghv = _dot(gs_r[...], wst_r[...]) + _dot(gd_r[...], wdt_r[...])
    y, inv = _ln_fwd(u_r[...])
    guv = _ln_bwd(ghv, y, inv)
    grv = _dot(guv, wn1t_r[...])
    gzv = jnp.where(z_r[...] > 0.0, grv, 0.0)
    o_r[...] = x_r[...] - DT * _dot(gzv, wn0t_r[...])


def _node_bwd(x, z, u, gs, gd, wst, wdt, wn1t, wn0t):
    grid = (N_NODES // BN,)
    row = pl.BlockSpec((BN, D), lambda i: (i, 0))
    w = pl.BlockSpec((D, D), lambda i: (0, 0))
    return pl.pallas_call(
        _node_bwd_body,
        grid=grid,
        in_specs=[row, row, row, row, row, w, w, w, w],
        out_specs=row,
        out_shape=jax.ShapeDtypeStruct((N_NODES, D), jnp.float32),
    )(x, z, u, gs, gd, wst, wdt, wn1t, wn0t)


MAXCH = 40
IDXBUF = MAXCH * CHUNK
GBUF = 3
SBUF = 3


def _sc_gather(a, b, src, dst):
    ne = src.shape[0]
    nch = ne // CHUNK
    mesh = plsc.VectorSubcoreMesh(core_axis_name="c", subcore_axis_name="s")
    out = jax.ShapeDtypeStruct((ne, D), jnp.float32)

    @functools.partial(
        pl.kernel,
        out_type=[out, out],
        mesh=mesh,
        scratch_types=[
            pltpu.VMEM((GBUF, CHUNK), jnp.int32),
            pltpu.VMEM((GBUF, CHUNK, D), jnp.float32),
            pltpu.VMEM_SHARED((N_NODES, D), jnp.float32),
        ] + [pltpu.SemaphoreType.DMA] * (3 * GBUF),
    )
    def k(a_h, b_h, src_h, dst_h, oa_h, ob_h, idx, rows, tbl, *sems):
        si = sems[:GBUF]
        sg = sems[GBUF:2 * GBUF]
        sw = sems[2 * GBUF:]
        c = lax.axis_index("c")
        s = lax.axis_index("s")

        rbase = s * 1000

        @pl.when(jnp.logical_and(c == 0, s < 10))
        def _():
            pltpu.sync_copy(a_h.at[pl.ds(rbase, 1000)],
                            tbl.at[pl.ds(rbase, 1000)])

        @pl.when(jnp.logical_and(c == 1, s < 10))
        def _():
            pltpu.sync_copy(b_h.at[pl.ds(rbase, 1000)],
                            tbl.at[pl.ds(rbase, 1000)])

        plsc.subcore_barrier()

        lo = (s * nch) // NS
        hi = ((s + 1) * nch) // NS

        def i_issue(cn, p):
            base = cn * CHUNK

            @pl.when(c == 0)
            def _():
                pltpu.async_copy(src_h.at[pl.ds(base, CHUNK)], idx.at[p],
                                 si[p])

            @pl.when(c == 1)
            def _():
                pltpu.async_copy(dst_h.at[pl.ds(base, CHUNK)], idx.at[p],
                                 si[p])

        def i_wait(p):
            pltpu.make_async_copy(src_h.at[pl.ds(0, CHUNK)], idx.at[p],
                                  si[p]).wait()

        def g_issue(p):
            pltpu.async_copy(tbl.at[idx.at[p]], rows.at[p], sg[p])

        def g_wait(p):
            pltpu.make_async_copy(tbl.at[idx.at[p]], rows.at[p],
                                  sg[p]).wait()

        def w_issue(cn, p):
            base = cn * CHUNK

            @pl.when(c == 0)
            def _():
                pltpu.async_copy(rows.at[p], oa_h.at[pl.ds(base, CHUNK)],
                                 sw[p])

            @pl.when(c == 1)
            def _():
                pltpu.async_copy(rows.at[p], ob_h.at[pl.ds(base, CHUNK)],
                                 sw[p])

        def w_wait(p):
            pltpu.make_async_copy(rows.at[p], oa_h.at[pl.ds(0, CHUNK)],
                                  sw[p]).wait()

        for p in range(GBUF):
            @pl.when(lo + p < hi)
            def _(p=p):
                i_issue(lo + p, p)

        def body(t, carry):
            cbase = lo + GBUF * t
            for p in range(GBUF):
                @pl.when(cbase + p < hi)
                def _(p=p):
                    i_wait(p)
                    g_issue(p)
            for p in range(GBUF):
                @pl.when(cbase + p < hi)
                def _(p=p, cc=cbase + p):
                    g_wait(p)
                    w_issue(cc, p)
            for p in range(GBUF):
                @pl.when(cbase + p < hi)
                def _(p=p):
                    w_wait(p)

                @pl.when(cbase + p + GBUF < hi)
                def _(p=p, cn=cbase + p + GBUF):
                    i_issue(cn, p)
            return carry

        nloc = (nch + NS - 1) // NS
        lax.fori_loop(0, (nloc + GBUF - 1) // GBUF, body, 0)

    return k(a, b, src, dst)


def _sc_scatter(gz, src, dst, init_s, init_d):
    ne = src.shape[0]
    nch = ne // CHUNK
    mesh = plsc.VectorSubcoreMesh(core_axis_name="c", subcore_axis_name="s")
    out = jax.ShapeDtypeStruct((N_NODES, D), jnp.float32)

    @functools.partial(
        pl.kernel,
        out_type=[out, out],
        mesh=mesh,
        scratch_types=[
            pltpu.VMEM((SBUF, CHUNK), jnp.int32),
            pltpu.VMEM((SBUF, CHUNK, D), jnp.float32),
            pltpu.VMEM_SHARED((N_NODES, D), jnp.float32),
        ] + [pltpu.SemaphoreType.DMA] * (2 * SBUF),
    )
    def k(gz_h, src_h, dst_h, is_h, id_h, os_h, od_h, idx, rows, table,
          *sems):
        sl = sems[:SBUF]
        ss = sems[SBUF:]
        c = lax.axis_index("c")
        s = lax.axis_index("s")
        rbase = s * 1000

        @pl.when(jnp.logical_and(c == 0, s < 10))
        def _():
            pltpu.sync_copy(is_h.at[pl.ds(rbase, 1000)],
                            table.at[pl.ds(rbase, 1000)])

        @pl.when(jnp.logical_and(c == 1, s < 10))
        def _():
            pltpu.sync_copy(id_h.at[pl.ds(rbase, 1000)],
                            table.at[pl.ds(rbase, 1000)])

        plsc.subcore_barrier()

        lo = (s * nch) // NS
        hi = ((s + 1) * nch) // NS

        def l_issue(cn, p):
            base = cn * CHUNK

            @pl.when(c == 0)
            def _():
                pltpu.async_copy(src_h.at[pl.ds(base, CHUNK)], idx.at[p],
                                 sl[p])

            @pl.when(c == 1)
            def _():
                pltpu.async_copy(dst_h.at[pl.ds(base, CHUNK)], idx.at[p],
                                 sl[p])

            pltpu.async_copy(gz_h.at[pl.ds(base, CHUNK)], rows.at[p], sl[p])

        def l_wait(p):
            pltpu.make_async_copy(src_h.at[pl.ds(0, CHUNK)], idx.at[p],
                                  sl[p]).wait()
            pltpu.make_async_copy(gz_h.at[pl.ds(0, CHUNK)], rows.at[p],
                                  sl[p]).wait()

        def s_issue(p):
            pltpu.async_copy(rows.at[p], table.at[idx.at[p]], ss[p], add=True)

        def s_wait(p):
            pltpu.make_async_copy(rows.at[p], table.at[idx.at[p]],
                                  ss[p]).wait()

        for p in range(SBUF):
            @pl.when(lo + p < hi)
            def _(p=p):
                l_issue(lo + p, p)

        def body(t, carry):
            cbase = lo + SBUF * t
            for p in range(SBUF):
                @pl.when(cbase + p < hi)
                def _(p=p):
                    l_wait(p)
                    s_issue(p)
            for p in range(SBUF):
                @pl.when(cbase + p < hi)
                def _(p=p):
                    s_wait(p)

                @pl.when(cbase + p + SBUF < hi)
                def _(p=p, cn=cbase + p + SBUF):
                    l_issue(cn, p)
            return carry

        lax.fori_loop(0, (nch // NS + 1 + SBUF) // SBUF, body, 0)
        plsc.subcore_barrier()

        @pl.when(jnp.logical_and(c == 0, s < 10))
        def _():
            pltpu.sync_copy(table.at[pl.ds(rbase, 1000)],
                            os_h.at[pl.ds(rbase, 1000)])

        @pl.when(jnp.logical_and(c == 1, s < 10))
        def _():
            pltpu.sync_copy(table.at[pl.ds(rbase, 1000)],
                            od_h.at[pl.ds(rbase, 1000)])

    return k(gz, src, dst, init_s, init_d)


def kernel(x, edge_attr, edge_index, enc_node_w0, enc_node_b0, enc_node_w1,
           enc_node_b1, enc_edge_w0, enc_edge_b0, enc_edge_w1, enc_edge_b1,
           mp_edge_w0, mp_edge_b0, mp_edge_w1, mp_edge_b1, mp_node_w0,
           mp_node_b0, mp_node_w1, mp_node_b1, dec_edge_w0, dec_edge_b0,
           dec_edge_w1, dec_edge_b1):
    src = edge_index[0].astype(jnp.int32)
    dst = edge_index[1].astype(jnp.int32)

    wsrc = mp_edge_w0[D:2 * D]
    wdst = mp_edge_w0[2 * D:3 * D]
    wme = mp_edge_w0[0:D]

    r1 = lambda v: v.reshape(1, D)
    a, b, u_v, z_v = _node_fwd(x, enc_node_w0, r1(enc_node_b0), enc_node_w1,
                               r1(enc_node_b1), wsrc, wdst)

    edge_w = (enc_edge_w0, r1(enc_edge_b0), enc_edge_w1, r1(enc_edge_b1),
              wme, r1(mp_edge_b0), mp_edge_w1, r1(mp_edge_b1), mp_edge_w1.T,
              dec_edge_w0, r1(dec_edge_b0), dec_edge_w0.T,
              dec_edge_w1[:, 0].reshape(1, D))

    gs = jnp.zeros((N_NODES, D), jnp.float32)
    gd = gs
    bounds = [0, 64000, 112000, 160000]
    for i in range(len(bounds) - 1):
        sl = slice(bounds[i], bounds[i + 1])
        ga_i, gb_i = _sc_gather(a, b, src[sl], dst[sl])
        gz_i = _edge_pipe(edge_attr[sl], ga_i, gb_i, *edge_w)
        gs, gd = _sc_scatter(gz_i, src[sl], dst[sl], gs, gd)

    return _node_bwd(x, z_v, u_v, gs, gd, wsrc.T, wdst.T,
                     enc_node_w1.T, enc_node_w0.T)
